# baseline (speedup 1.0000x reference)
import functools

import jax
import jax.numpy as jnp
from jax.experimental import pallas as pl
from jax.experimental.pallas import tpu as pltpu

TOPK = 30
NSLICE = 16
NCAND = 5
NEG_INF = float("-inf")


def _oddeven_merge(lo, n, r):
    step = r * 2
    if step < n:
        yield from _oddeven_merge(lo, n, step)
        yield from _oddeven_merge(lo + r, n, step)
        for i in range(lo + r, lo + n - r, step):
            yield (i, i + r)
    else:
        yield (lo, lo + r)


def _oddeven_merge_sort(lo, hi):
    if hi - lo >= 1:
        mid = lo + (hi - lo) // 2
        yield from _oddeven_merge_sort(lo, mid)
        yield from _oddeven_merge_sort(mid + 1, hi)
        yield from _oddeven_merge(lo, hi - lo + 1, 1)


def _prune_for_top(pairs, n_top):
    needed = set(range(n_top))
    kept = []
    for i, j in reversed(pairs):
        if i in needed or j in needed:
            kept.append((i, j))
            needed.add(i)
            needed.add(j)
    return list(reversed(kept))


_SORT_PAIRS = _prune_for_top(list(_oddeven_merge_sort(0, NSLICE - 1)), NCAND)


def _extract_kth_max_slices(slices, m, n_pulls):
    def step(_, carry):
        fr = carry[0]
        rest = list(carry[1:-1])
        mi = jnp.max(fr, axis=1, keepdims=True)
        popped = fr >= mi
        chain = [fr] + rest
        for a in range(len(chain) - 1):
            chain[a] = jnp.where(popped, chain[a + 1], chain[a])
        chain[-1] = jnp.where(popped, NEG_INF, chain[-1])
        return (*chain, mi)

    popped0 = slices[0] >= m
    chain = list(slices)
    for a in range(len(chain) - 1):
        chain[a] = jnp.where(popped0, chain[a + 1], chain[a])
    chain[-1] = jnp.where(popped0, NEG_INF, chain[-1])

    out = jax.lax.fori_loop(0, n_pulls - 1, step, (*chain, m), unroll=29)
    return out[-1]


def _extract_kth_max_multiset(slices, m, n_pulls):
    def step(_, carry):
        t = carry[-1]
        cur = list(carry[:-1])
        red = cur[0]
        for c in cur[1:]:
            red = jnp.maximum(red, c)
        mi = jnp.max(red, axis=1, keepdims=True)
        cur = [jnp.where(c >= mi, NEG_INF, c) for c in cur]
        return (*cur, mi)

    cur0 = tuple(jnp.where(c >= m, NEG_INF, c) for c in slices)
    out = jax.lax.fori_loop(0, n_pulls - 1, step, (*cur0, m), unroll=29)
    return out[-1]


def _attn_block_kernel(q_ref, k_ref, v_ref, attn_ref, ctx_ref, t_ref, z_ref):
    qb = q_ref[0]
    kb = k_ref[0]
    s = jax.lax.dot_general(
        qb, kb, (((1,), (1,)), ((), ())), preferred_element_type=jnp.float32
    )
    S = s.shape[1]
    w = S // NSLICE

    sl = [s[:, i * w : (i + 1) * w] for i in range(NSLICE)]
    for i, j in _SORT_PAIRS:
        hi = jnp.maximum(sl[i], sl[j])
        lo = jnp.minimum(sl[i], sl[j])
        sl[i], sl[j] = hi, lo

    m = jnp.max(sl[0], axis=1, keepdims=True)

    t_cand = _extract_kth_max_slices(sl[:NCAND], m, TOPK)

    c_gt = jnp.sum((s > t_cand).astype(jnp.float32), axis=1, keepdims=True)
    c_ge = jnp.sum((s >= t_cand).astype(jnp.float32), axis=1, keepdims=True)
    ok = jnp.logical_and(c_gt < TOPK, c_ge == TOPK)
    t_ref[...] = t_cand
    zc = jnp.zeros_like(m)
    for c in sl[:NCAND]:
        zc = zc + jnp.sum(
            jnp.where(c >= t_cand, jnp.exp(c - m), 0.0), axis=1, keepdims=True
        )
    z_ref[...] = zc

    @pl.when(jnp.logical_not(jnp.all(ok)))
    def _fallback():
        t_full = _extract_kth_max_multiset(sl, m, TOPK)
        t_ref[...] = t_full
        z_ref[...] = jnp.sum(
            jnp.where(s >= t_full, jnp.exp(s - m), 0.0), axis=1, keepdims=True
        )

    t = t_ref[...]
    invz = 1.0 / z_ref[...]
    attn = jnp.where(s >= t, jnp.exp(s - m) * invz, 0.0)
    attn_ref[0] = attn
    ctx_ref[0] = jax.lax.dot_general(
        attn, v_ref[0], (((1,), (0,)), ((), ())), preferred_element_type=jnp.float32
    )


@functools.partial(jax.jit, static_argnames=("interpret",))
def _run(q, k, v, interpret=False):
    bh, S, d = q.shape
    blk = min(512, S)
    grid = (bh, S // blk)
    attn, ctx = pl.pallas_call(
        _attn_block_kernel,
        grid=grid,
        in_specs=[
            pl.BlockSpec((1, blk, d), lambda h, i: (h, i, 0)),
            pl.BlockSpec((1, S, d), lambda h, i: (h, 0, 0)),
            pl.BlockSpec((1, S, d), lambda h, i: (h, 0, 0)),
        ],
        out_specs=[
            pl.BlockSpec((1, blk, S), lambda h, i: (h, i, 0)),
            pl.BlockSpec((1, blk, d), lambda h, i: (h, i, 0)),
        ],
        out_shape=[
            jax.ShapeDtypeStruct((bh, S, S), jnp.float32),
            jax.ShapeDtypeStruct((bh, S, d), jnp.float32),
        ],
        scratch_shapes=[
            pltpu.VMEM((blk, 1), jnp.float32),
            pltpu.VMEM((blk, 1), jnp.float32),
        ],
        compiler_params=pltpu.CompilerParams(
            dimension_semantics=("parallel", "arbitrary"),
        ),
        interpret=interpret,
    )(q, k, v)
    return ctx, attn


def kernel(q, k, v, B, num_heads):
    return _run(q, k, v)


# speedup vs baseline: 1.0089x; 1.0089x over previous
import functools

import jax
import jax.numpy as jnp
from jax.experimental import pallas as pl
from jax.experimental.pallas import tpu as pltpu

TOPK = 30
NSLICE = 16
NCAND = 5
NEG_INF = float("-inf")


def _oddeven_merge(lo, n, r):
    step = r * 2
    if step < n:
        yield from _oddeven_merge(lo, n, step)
        yield from _oddeven_merge(lo + r, n, step)
        for i in range(lo + r, lo + n - r, step):
            yield (i, i + r)
    else:
        yield (lo, lo + r)


def _oddeven_merge_sort(lo, hi):
    if hi - lo >= 1:
        mid = lo + (hi - lo) // 2
        yield from _oddeven_merge_sort(lo, mid)
        yield from _oddeven_merge_sort(mid + 1, hi)
        yield from _oddeven_merge(lo, hi - lo + 1, 1)


def _prune_for_top(pairs, n_top):
    needed = set(range(n_top))
    kept = []
    for i, j in reversed(pairs):
        if i in needed or j in needed:
            kept.append((i, j))
            needed.add(i)
            needed.add(j)
    return list(reversed(kept))


_SORT_PAIRS = _prune_for_top(list(_oddeven_merge_sort(0, NSLICE - 1)), NCAND)


def _extract_kth_max_slices(slices, m, n_pulls):
    def step(_, carry):
        fr = carry[0]
        rest = list(carry[1:-1])
        mi = jnp.max(fr, axis=1, keepdims=True)
        popped = fr >= mi
        chain = [fr] + rest
        for a in range(len(chain) - 1):
            chain[a] = jnp.where(popped, chain[a + 1], chain[a])
        chain[-1] = jnp.where(popped, NEG_INF, chain[-1])
        return (*chain, mi)

    popped0 = slices[0] >= m
    chain = list(slices)
    for a in range(len(chain) - 1):
        chain[a] = jnp.where(popped0, chain[a + 1], chain[a])
    chain[-1] = jnp.where(popped0, NEG_INF, chain[-1])

    out = jax.lax.fori_loop(0, n_pulls - 1, step, (*chain, m), unroll=29)
    return out[-1]


def _extract_kth_max_multiset(slices, m, n_pulls):
    def step(_, carry):
        t = carry[-1]
        cur = list(carry[:-1])
        red = cur[0]
        for c in cur[1:]:
            red = jnp.maximum(red, c)
        mi = jnp.max(red, axis=1, keepdims=True)
        cur = [jnp.where(c >= mi, NEG_INF, c) for c in cur]
        return (*cur, mi)

    cur0 = tuple(jnp.where(c >= m, NEG_INF, c) for c in slices)
    out = jax.lax.fori_loop(0, n_pulls - 1, step, (*cur0, m), unroll=29)
    return out[-1]


def _attn_block_kernel(q_ref, k_ref, v_ref, attn_ref, ctx_ref, t_ref):
    qb = q_ref[0]
    kb = k_ref[0]
    s = jax.lax.dot_general(
        qb, kb, (((1,), (1,)), ((), ())), preferred_element_type=jnp.float32
    )
    S = s.shape[1]
    w = S // NSLICE

    sl = [s[:, i * w : (i + 1) * w] for i in range(NSLICE)]
    for i, j in _SORT_PAIRS:
        hi = jnp.maximum(sl[i], sl[j])
        lo = jnp.minimum(sl[i], sl[j])
        sl[i], sl[j] = hi, lo

    m = jnp.max(sl[0], axis=1, keepdims=True)

    t_cand = _extract_kth_max_slices(sl[:NCAND], m, TOPK)

    c_gt = jnp.sum((s > t_cand).astype(jnp.float32), axis=1, keepdims=True)
    ok = c_gt < TOPK
    t_ref[...] = t_cand

    @pl.when(jnp.logical_not(jnp.all(ok)))
    def _fallback():
        t_ref[...] = _extract_kth_max_multiset(sl, m, TOPK)

    t = t_ref[...]
    wexp = jnp.where(s >= t, jnp.exp(s - m), 0.0)
    z = jnp.sum(wexp, axis=1, keepdims=True)
    attn = wexp * (1.0 / z)
    attn_ref[0] = attn
    ctx_ref[0] = jax.lax.dot_general(
        attn, v_ref[0], (((1,), (0,)), ((), ())), preferred_element_type=jnp.float32
    )


@functools.partial(jax.jit, static_argnames=("interpret",))
def _run(q, k, v, interpret=False):
    bh, S, d = q.shape
    blk = min(512, S)
    grid = (bh, S // blk)
    attn, ctx = pl.pallas_call(
        _attn_block_kernel,
        grid=grid,
        in_specs=[
            pl.BlockSpec((1, blk, d), lambda h, i: (h, i, 0)),
            pl.BlockSpec((1, S, d), lambda h, i: (h, 0, 0)),
            pl.BlockSpec((1, S, d), lambda h, i: (h, 0, 0)),
        ],
        out_specs=[
            pl.BlockSpec((1, blk, S), lambda h, i: (h, i, 0)),
            pl.BlockSpec((1, blk, d), lambda h, i: (h, i, 0)),
        ],
        out_shape=[
            jax.ShapeDtypeStruct((bh, S, S), jnp.float32),
            jax.ShapeDtypeStruct((bh, S, d), jnp.float32),
        ],
        scratch_shapes=[pltpu.VMEM((blk, 1), jnp.float32)],
        compiler_params=pltpu.CompilerParams(
            dimension_semantics=("parallel", "arbitrary"),
        ),
        interpret=interpret,
    )(q, k, v)
    return ctx, attn


def kernel(q, k, v, B, num_heads):
    return _run(q, k, v)
